# baseline (speedup 1.0000x reference)
import functools

import jax
import jax.numpy as jnp
from jax import lax
from jax.experimental import pallas as pl
from jax.experimental.pallas import tpu as pltpu
from jax.experimental.pallas import tpu_sc as plsc

N_SEG = 4096
H = 512
W = 512
N_PTS = H * W
N_CH = 64
L = 16
ROWS = 16
CHUNK = ROWS * W
N_CHUNKS = N_PTS // CHUNK
PR = H // 16


def _sc_segment_mean(vals4d, map2d):
  mesh = plsc.VectorSubcoreMesh(core_axis_name="c", subcore_axis_name="s")

  @functools.partial(
      pl.kernel,
      out_type=jax.ShapeDtypeStruct((N_CH * N_SEG,), jnp.float32),
      mesh=mesh,
      compiler_params=pltpu.CompilerParams(
          needs_layout_passes=False, use_tc_tiling_on_sc=True),
      scratch_types=[
          pltpu.VMEM((4, ROWS, W), jnp.int32),
          pltpu.VMEM((4, ROWS, W), jnp.float32),
          pltpu.VMEM((4, ROWS, W), jnp.float32),
          pltpu.VMEM((N_SEG,), jnp.float32),
          pltpu.VMEM((N_SEG,), jnp.float32),
          pltpu.VMEM((N_SEG,), jnp.float32),
          pltpu.VMEM((N_SEG,), jnp.float32),
          pltpu.VMEM_SHARED((16 * N_SEG,), jnp.float32),
          pltpu.VMEM_SHARED((N_SEG,), jnp.float32),
          pltpu.SemaphoreType.DMA,
          pltpu.SemaphoreType.DMA,
          pltpu.SemaphoreType.DMA,
          pltpu.SemaphoreType.DMA,
      ],
  )
  def body(vals_hbm, map_hbm, out_hbm, idx_v, v0_v, v1_v, acc0, acc1,
           cnt_acc, cnt_rd, cnt_sh, inv_sh, semA, semB, semC, semD):
    cid = lax.axis_index("c")
    sid = lax.axis_index("s")
    wid = sid * 2 + cid

    zz = jnp.zeros((L,), jnp.float32)

    @plsc.parallel_loop(0, N_SEG // L, unroll=4)
    def zero_body(i):
      d = pl.ds(i * L, L)
      acc0[d] = zz
      acc1[d] = zz
      cnt_acc[d] = zz

    ones = jnp.ones((L,), jnp.float32)
    for h in range(PR // ROWS):
      pltpu.sync_copy(
          map_hbm.at[pl.ds(sid * PR + h * ROWS, ROWS), :], idx_v.at[0])

      @plsc.parallel_loop(0, CHUNK // L, unroll=8)
      def cnt_body(i):
        r = lax.shift_right_logical(i, 5)
        col = lax.shift_left(jnp.bitwise_and(i, 31), 4)
        idx = idx_v[0, r, pl.ds(col, L)]
        plsc.addupdate_scatter(cnt_acc, [idx], ones)

    pltpu.sync_copy(cnt_acc, cnt_sh.at[pl.ds(sid * N_SEG, N_SEG)])
    plsc.subcore_barrier()

    ch0 = wid * 2
    a0 = lax.shift_right_logical(ch0, 3)
    b0 = jnp.bitwise_and(ch0, 7)
    a1 = lax.shift_right_logical(ch0 + 1, 3)
    b1 = jnp.bitwise_and(ch0 + 1, 7)
    sems = (semA, semB, semC, semD)

    def chunk_copies(k, b):
      rows = pl.ds(k * ROWS, ROWS)
      return (
          (map_hbm.at[rows, :], idx_v.at[b]),
          (vals_hbm.at[a0, b0, rows, :], v0_v.at[b]),
          (vals_hbm.at[a1, b1, rows, :], v1_v.at[b]),
      )

    def start(k, b):
      for src, dst in chunk_copies(k, b):
        pltpu.async_copy(src, dst, sems[b])

    def wait(k, b):
      for src, dst in chunk_copies(k, b):
        pltpu.make_async_copy(src, dst, sems[b]).wait()

    def process(b):
      @plsc.parallel_loop(0, CHUNK // L, unroll=16)
      def pt_body(i):
        r = lax.shift_right_logical(i, 5)
        col = lax.shift_left(jnp.bitwise_and(i, 31), 4)
        d = pl.ds(col, L)
        idx = idx_v[b, r, d]
        plsc.addupdate_scatter(acc0, [idx], v0_v[b, r, d])
        plsc.addupdate_scatter(acc1, [idx], v1_v[b, r, d])

    for b in range(4):
      start(b, b)

    def quad_body(p, carry):
      k = 4 * p
      for b in range(4):
        wait(k + b, b)
        process(b)
        start(k + 4 + b, b)
      return carry

    lax.fori_loop(0, N_CHUNKS // 4 - 1, quad_body, 0)
    for b in range(4):
      wait(N_CHUNKS - 4 + b, b)
      process(b)

    SEG_B = N_SEG // 16
    for r in range(16):
      pltpu.sync_copy(cnt_sh.at[pl.ds(r * N_SEG + sid * SEG_B, SEG_B)],
                      cnt_rd.at[pl.ds(r * SEG_B, SEG_B)])

    one = jnp.ones((L,), jnp.float32)

    @plsc.parallel_loop(0, SEG_B // L, unroll=4)
    def inv_body(i):
      dl = pl.ds(i * L, L)
      tot = cnt_rd[dl]
      for r in range(1, 16):
        tot = tot + cnt_rd[pl.ds(r * SEG_B + i * L, L)]
      cnt_rd[dl] = one / jnp.maximum(tot, one)

    pltpu.sync_copy(cnt_rd.at[pl.ds(0, SEG_B)], inv_sh.at[pl.ds(sid * SEG_B, SEG_B)])
    plsc.subcore_barrier()
    pltpu.sync_copy(inv_sh, cnt_rd)

    @plsc.parallel_loop(0, N_SEG // L, unroll=4)
    def div_body(i):
      dl = pl.ds(i * L, L)
      inv = cnt_rd[dl]
      acc0[dl] = acc0[dl] * inv
      acc1[dl] = acc1[dl] * inv

    pltpu.sync_copy(acc0, out_hbm.at[pl.ds(ch0 * N_SEG, N_SEG)])
    pltpu.sync_copy(acc1, out_hbm.at[pl.ds((ch0 + 1) * N_SEG, N_SEG)])

  return body(vals4d, map2d)


@jax.jit
def kernel(output, mapping):
  out = _sc_segment_mean(output, mapping.reshape(H, W))
  return out.reshape(8, 8, N_SEG)


# speedup vs baseline: 1.0188x; 1.0188x over previous
import functools

import jax
import jax.numpy as jnp
from jax import lax
from jax.experimental import pallas as pl
from jax.experimental.pallas import tpu as pltpu
from jax.experimental.pallas import tpu_sc as plsc

N_SEG = 4096
H = 512
W = 512
N_PTS = H * W
N_CH = 64
L = 16
ROWS = 32
CHUNK = ROWS * W
N_CHUNKS = N_PTS // CHUNK
PR = H // 16


def _sc_segment_mean(vals4d, map2d):
  mesh = plsc.VectorSubcoreMesh(core_axis_name="c", subcore_axis_name="s")

  @functools.partial(
      pl.kernel,
      out_type=jax.ShapeDtypeStruct((N_CH * N_SEG,), jnp.float32),
      mesh=mesh,
      compiler_params=pltpu.CompilerParams(
          needs_layout_passes=False, use_tc_tiling_on_sc=True),
      scratch_types=[
          pltpu.VMEM((2, ROWS, W), jnp.int32),
          pltpu.VMEM((2, ROWS, W), jnp.float32),
          pltpu.VMEM((2, ROWS, W), jnp.float32),
          pltpu.VMEM((N_SEG,), jnp.float32),
          pltpu.VMEM((N_SEG,), jnp.float32),
          pltpu.VMEM((N_SEG,), jnp.float32),
          pltpu.VMEM((N_SEG,), jnp.float32),
          pltpu.VMEM_SHARED((16 * N_SEG,), jnp.float32),
          pltpu.VMEM_SHARED((N_SEG,), jnp.float32),
          pltpu.SemaphoreType.DMA,
          pltpu.SemaphoreType.DMA,
      ],
  )
  def body(vals_hbm, map_hbm, out_hbm, idx_v, v0_v, v1_v, acc0, acc1,
           cnt_acc, cnt_rd, cnt_sh, inv_sh, semA, semB):
    cid = lax.axis_index("c")
    sid = lax.axis_index("s")
    wid = sid * 2 + cid

    zz = jnp.zeros((L,), jnp.float32)

    @plsc.parallel_loop(0, N_SEG // L, unroll=4)
    def zero_body(i):
      d = pl.ds(i * L, L)
      acc0[d] = zz
      acc1[d] = zz
      cnt_acc[d] = zz

    ones = jnp.ones((L,), jnp.float32)
    pltpu.sync_copy(map_hbm.at[pl.ds(sid * PR, PR), :], idx_v.at[0])

    @plsc.parallel_loop(0, CHUNK // L, unroll=8)
    def cnt_body(i):
      r = lax.shift_right_logical(i, 5)
      col = lax.shift_left(jnp.bitwise_and(i, 31), 4)
      idx = idx_v[0, r, pl.ds(col, L)]
      plsc.addupdate_scatter(cnt_acc, [idx], ones)

    pltpu.sync_copy(cnt_acc, cnt_sh.at[pl.ds(sid * N_SEG, N_SEG)])
    plsc.subcore_barrier()

    ch0 = wid * 2
    a0 = lax.shift_right_logical(ch0, 3)
    b0 = jnp.bitwise_and(ch0, 7)
    a1 = lax.shift_right_logical(ch0 + 1, 3)
    b1 = jnp.bitwise_and(ch0 + 1, 7)
    sems = (semA, semB)

    def chunk_copies(k, b):
      rows = pl.ds(k * ROWS, ROWS)
      return (
          (map_hbm.at[rows, :], idx_v.at[b]),
          (vals_hbm.at[a0, b0, rows, :], v0_v.at[b]),
          (vals_hbm.at[a1, b1, rows, :], v1_v.at[b]),
      )

    def start(k, b):
      for src, dst in chunk_copies(k, b):
        pltpu.async_copy(src, dst, sems[b])

    def wait(k, b):
      for src, dst in chunk_copies(k, b):
        pltpu.make_async_copy(src, dst, sems[b]).wait()

    def process(b):
      @plsc.parallel_loop(0, CHUNK // L, unroll=8)
      def pt_body(i):
        r = lax.shift_right_logical(i, 5)
        col = lax.shift_left(jnp.bitwise_and(i, 31), 4)
        d = pl.ds(col, L)
        idx = idx_v[b, r, d]
        plsc.addupdate_scatter(acc0, [idx], v0_v[b, r, d])
        plsc.addupdate_scatter(acc1, [idx], v1_v[b, r, d])

    start(0, 0)
    start(1, 1)

    def pair_body(p, carry):
      k = 2 * p
      wait(k, 0)
      process(0)
      start(k + 2, 0)
      wait(k + 1, 1)
      process(1)
      start(k + 3, 1)
      return carry

    lax.fori_loop(0, N_CHUNKS // 2 - 1, pair_body, 0)
    wait(N_CHUNKS - 2, 0)
    process(0)
    wait(N_CHUNKS - 1, 1)
    process(1)

    SEG_B = N_SEG // 16
    for r in range(16):
      pltpu.sync_copy(cnt_sh.at[pl.ds(r * N_SEG + sid * SEG_B, SEG_B)],
                      cnt_rd.at[pl.ds(r * SEG_B, SEG_B)])

    one = jnp.ones((L,), jnp.float32)

    @plsc.parallel_loop(0, SEG_B // L, unroll=4)
    def inv_body(i):
      dl = pl.ds(i * L, L)
      tot = cnt_rd[dl]
      for r in range(1, 16):
        tot = tot + cnt_rd[pl.ds(r * SEG_B + i * L, L)]
      cnt_rd[dl] = one / jnp.maximum(tot, one)

    pltpu.sync_copy(cnt_rd.at[pl.ds(0, SEG_B)], inv_sh.at[pl.ds(sid * SEG_B, SEG_B)])
    plsc.subcore_barrier()
    pltpu.sync_copy(inv_sh, cnt_rd)

    @plsc.parallel_loop(0, N_SEG // L, unroll=4)
    def div_body(i):
      dl = pl.ds(i * L, L)
      inv = cnt_rd[dl]
      acc0[dl] = acc0[dl] * inv
      acc1[dl] = acc1[dl] * inv

    pltpu.sync_copy(acc0, out_hbm.at[pl.ds(ch0 * N_SEG, N_SEG)])
    pltpu.sync_copy(acc1, out_hbm.at[pl.ds((ch0 + 1) * N_SEG, N_SEG)])

  return body(vals4d, map2d)


@jax.jit
def kernel(output, mapping):
  out = _sc_segment_mean(output, mapping.reshape(H, W))
  return out.reshape(8, 8, N_SEG)
